# baseline (speedup 1.0000x reference)
import functools

import jax
import jax.numpy as jnp
from jax import lax
from jax.experimental import pallas as pl
from jax.experimental.pallas import tpu as pltpu
from jax.experimental.pallas import tpu_sc as plsc

B = 16384
D = 64
LANES = 16
PAD = 17

_info = plsc.get_sparse_core_info()
NC = _info.num_cores
NS = _info.num_subcores
NW = NC * NS
BPW = B // NW
NCHUNK = 4
CHUNK = BPW // NCHUNK

_mesh = plsc.VectorSubcoreMesh(core_axis_name="c", subcore_axis_name="s")

_HI = jnp.int32(-65536)


def _bf16_halves(ref, r, lo):
    x = ref[r, pl.ds(lo, LANES)]
    a = plsc.bitcast(x << 16, jnp.float32)
    b = plsc.bitcast(x & _HI, jnp.float32)
    return a, b


@functools.partial(
    pl.kernel,
    out_type=jax.ShapeDtypeStruct((B,), jnp.float32),
    mesh=_mesh,
    compiler_params=pltpu.CompilerParams(
        needs_layout_passes=False, use_tc_tiling_on_sc=False),
    scratch_types=[
        pltpu.VMEM((NCHUNK, CHUNK), jnp.int32),
        pltpu.VMEM((NCHUNK, CHUNK), jnp.int32),
        pltpu.VMEM((BPW, D // 2), jnp.int32),
        pltpu.VMEM((BPW, D // 2), jnp.int32),
        pltpu.VMEM((BPW * PAD,), jnp.float32),
        pltpu.VMEM((BPW,), jnp.float32),
        pltpu.SemaphoreType.DMA,
        pltpu.SemaphoreType.DMA,
    ],
)
def _sc_dot(user_hbm, item_hbm, uemb_hbm, iemb_hbm, out_hbm,
            uidx, iidx, urows, irows, part, outc, usem, isem):
    wid = lax.axis_index("s") * NC + lax.axis_index("c")
    base = wid * BPW

    pltpu.sync_copy(user_hbm.at[wid], uidx)
    pltpu.sync_copy(item_hbm.at[wid], iidx)

    copies = []
    for c in range(NCHUNK):
        copies.append(pltpu.async_copy(
            uemb_hbm.at[uidx.at[c]], urows.at[pl.ds(c * CHUNK, CHUNK)], usem))
        copies.append(pltpu.async_copy(
            iemb_hbm.at[iidx.at[c]], irows.at[pl.ds(c * CHUNK, CHUNK)], isem))
    for cp in copies:
        cp.wait()

    def row_body(r, carry):
        ua, ub = _bf16_halves(urows, r, 0)
        va, vb = _bf16_halves(irows, r, 0)
        acc = ua * va + ub * vb
        ua, ub = _bf16_halves(urows, r, LANES)
        va, vb = _bf16_halves(irows, r, LANES)
        acc += ua * va + ub * vb
        part[pl.ds(r * PAD, LANES)] = acc
        return carry

    lax.fori_loop(0, BPW, row_body, 0, unroll=2)

    def grp_body(g, carry):
        rows = (g * LANES + lax.iota(jnp.int32, LANES)) * PAD
        acc = plsc.load_gather(part, [rows])
        for j in range(1, LANES):
            acc += plsc.load_gather(part, [rows + j])
        outc[pl.ds(g * LANES, LANES)] = acc
        return carry

    lax.fori_loop(0, BPW // LANES, grp_body, 0, unroll=2)

    pltpu.sync_copy(outc, out_hbm.at[pl.ds(base, BPW)])


def kernel(user, item, attr, user_embedding, item_embedding):
    del attr
    user = user.astype(jnp.int32).reshape(NW, NCHUNK, CHUNK)
    item = item.astype(jnp.int32).reshape(NW, NCHUNK, CHUNK)
    uemb = jax.lax.bitcast_convert_type(
        user_embedding.astype(jnp.bfloat16).reshape(1000000, D // 2, 2),
        jnp.int32)
    iemb = jax.lax.bitcast_convert_type(
        item_embedding.astype(jnp.bfloat16).reshape(1000000, D // 2, 2),
        jnp.int32)
    return _sc_dot(user, item, uemb, iemb)


# speedup vs baseline: 2.2064x; 2.2064x over previous
import functools

import jax
import jax.numpy as jnp
from jax import lax
from jax.experimental import pallas as pl
from jax.experimental.pallas import tpu as pltpu
from jax.experimental.pallas import tpu_sc as plsc

B = 16384
D = 64
LANES = 16
PAD = 17

_info = plsc.get_sparse_core_info()
NC = _info.num_cores
NS = _info.num_subcores
NW = NC * NS
BPW = B // NW
CH = 32
NCH = BPW // CH

_mesh = plsc.VectorSubcoreMesh(core_axis_name="c", subcore_axis_name="s")

_HI = jnp.int32(-65536)


def _bf16_halves(ref, r, lo):
    x = plsc.bitcast(ref[r, pl.ds(lo, 2 * LANES)], jnp.int32)
    a = plsc.bitcast(x << 16, jnp.float32)
    b = plsc.bitcast(x & _HI, jnp.float32)
    return a, b


@functools.partial(
    pl.kernel,
    out_type=jax.ShapeDtypeStruct((B,), jnp.float32),
    mesh=_mesh,
    compiler_params=pltpu.CompilerParams(
        needs_layout_passes=False, use_tc_tiling_on_sc=False),
    scratch_types=[
        pltpu.VMEM((BPW,), jnp.int32),
        pltpu.VMEM((BPW,), jnp.int32),
        pltpu.VMEM((CH, D), jnp.bfloat16),
        pltpu.VMEM((CH, D), jnp.bfloat16),
        pltpu.VMEM((BPW * PAD,), jnp.float32),
        pltpu.VMEM((BPW,), jnp.float32),
        pltpu.SemaphoreType.DMA,
        pltpu.SemaphoreType.DMA,
    ],
)
def _sc_dot(user_hbm, item_hbm, uemb_hbm, iemb_hbm, out_hbm,
            uidx, iidx, urows, irows, part, outc, usem, isem):
    wid = lax.axis_index("s") * NC + lax.axis_index("c")
    base = wid * BPW

    pltpu.sync_copy(user_hbm.at[pl.ds(base, BPW)], uidx)
    pltpu.sync_copy(item_hbm.at[pl.ds(base, BPW)], iidx)

    def chunk_body(g, carry):
        descs = []
        for k2 in range(CH // LANES):
            uvec = uidx[pl.ds(g * CH + k2 * LANES, LANES)]
            ivec = iidx[pl.ds(g * CH + k2 * LANES, LANES)]
            for j in range(LANES):
                e2 = k2 * LANES + j
                descs.append(pltpu.async_copy(
                    uemb_hbm.at[pl.ds(uvec[j], 1)],
                    urows.at[pl.ds(e2, 1)], usem))
                descs.append(pltpu.async_copy(
                    iemb_hbm.at[pl.ds(ivec[j], 1)],
                    irows.at[pl.ds(e2, 1)], isem))
        for dsc in descs:
            dsc.wait()

        def row_body(r, carry2):
            ua, ub = _bf16_halves(urows, r, 0)
            va, vb = _bf16_halves(irows, r, 0)
            acc = ua * va + ub * vb
            ua, ub = _bf16_halves(urows, r, 2 * LANES)
            va, vb = _bf16_halves(irows, r, 2 * LANES)
            acc += ua * va + ub * vb
            part[pl.ds((g * CH + r) * PAD, LANES)] = acc
            return carry2

        lax.fori_loop(0, CH, row_body, 0, unroll=2)
        return carry

    lax.fori_loop(0, NCH, chunk_body, 0)

    def grp_body(g, carry):
        rows = (g * LANES + lax.iota(jnp.int32, LANES)) * PAD
        acc = plsc.load_gather(part, [rows])
        for j in range(1, LANES):
            acc += plsc.load_gather(part, [rows + j])
        outc[pl.ds(g * LANES, LANES)] = acc
        return carry

    lax.fori_loop(0, BPW // LANES, grp_body, 0, unroll=2)

    pltpu.sync_copy(outc, out_hbm.at[pl.ds(base, BPW)])


def kernel(user, item, attr, user_embedding, item_embedding):
    del attr
    return _sc_dot(user.astype(jnp.int32), item.astype(jnp.int32),
                   user_embedding.astype(jnp.bfloat16),
                   item_embedding.astype(jnp.bfloat16))
